# baseline (speedup 1.0000x reference)
import jax
import jax.numpy as jnp
from jax import lax
from jax.experimental import pallas as pl
from jax.experimental.pallas import tpu as pltpu
from jax.experimental.pallas import tpu_sc as plsc

B0, B1, N = 4, 4096, 2048
ROWS = B0 * B1
K = (N - 1) // 2
NW = 32
ROWS_PER_W = ROWS // NW
RB = 32
L = 16
UNROLL = 8
S1 = 8


def _median_body(x_hbm, out_hbm, x_vmem, buf_vmem, out_vmem):
    c = lax.axis_index("c")
    s = lax.axis_index("s")
    wid = s * 2 + c
    row0 = wid * ROWS_PER_W

    kvec = jnp.full((L,), K, jnp.int32)
    zero = jnp.zeros((L,), jnp.int32)
    one = jnp.ones((L,), jnp.int32)
    top_bit = jnp.full((L,), -(2 ** 31), jnp.int32)
    mant_mask = jnp.full((L,), 0x7FFFFFFF, jnp.int32)
    pref_mask = jnp.full((L,), -(2 ** (32 - S1)), jnp.int32)
    nan_vec = jnp.full((L,), jnp.nan, jnp.float32)
    lane = lax.iota(jnp.int32, L)
    lane0 = lane == zero

    def key_to_float(key):
        fbits = jnp.where(key < zero, key & mant_mask, ~key)
        return lax.bitcast_convert_type(fbits, jnp.float32)

    def block_body(blk, carry):
        base_row = blk * RB
        pltpu.sync_copy(x_hbm.at[pl.ds((row0 + base_row) * N, RB * N)], x_vmem)

        def row_body(r, carry):
            off = r * N

            def bit1(_, st):
                prefix, bitv, below = st
                cand = prefix | bitv
                t = key_to_float(cand)

                def chunk(j, cnt):
                    e = off + j * (UNROLL * L)
                    for u in range(UNROLL):
                        v = x_vmem[pl.ds(e + u * L, L)]
                        cnt = cnt + plsc.all_reduce_population_count(v < t)
                    return cnt

                cnt = lax.fori_loop(0, N // (UNROLL * L), chunk, zero)
                acc = cnt <= kvec
                return (jnp.where(acc, cand, prefix),
                        lax.shift_right_logical(bitv, 1),
                        jnp.where(acc, cnt, below))

            prefix, bitv, below = lax.fori_loop(0, S1, bit1,
                                                (zero, top_bit, zero))
            krem = kvec - below

            def comp(j, o):
                e = off + j * L
                v = x_vmem[pl.ds(e, L)]
                b = lax.bitcast_convert_type(v + jnp.float32(0.0), jnp.int32)
                key = jnp.where(b < zero, ~b, b | top_bit)
                m = (key & pref_mask) == prefix
                csum = plsc.cumsum(jnp.where(m, one, zero))
                idx = jnp.where(m, o + csum - one, zero)
                plsc.store_scatter(buf_vmem, [idx], v, mask=m)
                return o + plsc.all_reduce_population_count(m)

            o = lax.fori_loop(0, N // L, comp, zero)
            plsc.store_scatter(buf_vmem, [o + lane], nan_vec)
            nvreg = lax.shift_right_logical(jnp.max(o) + (L - 1), 4)

            def bit2(_, st):
                prefix, bitv = st
                cand = prefix | bitv
                t = key_to_float(cand)

                def chunk2(j, cnt):
                    v = buf_vmem[pl.ds(j * L, L)]
                    return cnt + plsc.all_reduce_population_count(v < t)

                cnt = lax.fori_loop(0, nvreg, chunk2, zero)
                return (jnp.where(cnt <= krem, cand, prefix),
                        lax.shift_right_logical(bitv, 1))

            prefix, _ = lax.fori_loop(0, 32 - S1, bit2, (prefix, bitv))
            val = key_to_float(prefix)
            plsc.store_scatter(out_vmem,
                               [jnp.full((L,), base_row + r, jnp.int32)],
                               val, mask=lane0)
            return carry

        return lax.fori_loop(0, RB, row_body, carry)

    lax.fori_loop(0, ROWS_PER_W // RB, block_body, 0)
    pltpu.sync_copy(out_vmem, out_hbm.at[pl.ds(row0, ROWS_PER_W)])


def _median_call(xflat):
    return pl.kernel(
        _median_body,
        out_type=jax.ShapeDtypeStruct((ROWS,), jnp.float32),
        mesh=plsc.VectorSubcoreMesh(core_axis_name="c", subcore_axis_name="s"),
        scratch_types=[
            pltpu.VMEM((RB * N,), jnp.float32),
            pltpu.VMEM((N + L,), jnp.float32),
            pltpu.VMEM((ROWS_PER_W,), jnp.float32),
        ],
        compiler_params=pltpu.CompilerParams(needs_layout_passes=False),
    )(xflat)


@jax.jit
def kernel(x):
    out = _median_call(x.reshape(ROWS * N))
    return out.reshape(B0, B1)


# speedup vs baseline: 1.1331x; 1.1331x over previous
import jax
import jax.numpy as jnp
from jax import lax
from jax.experimental import pallas as pl
from jax.experimental.pallas import tpu as pltpu
from jax.experimental.pallas import tpu_sc as plsc

B0, B1, N = 4, 4096, 2048
ROWS = B0 * B1
K = (N - 1) // 2
NW = 32
ROWS_PER_W = ROWS // NW
RB = 32
L = 16
UNROLL = 8
S1 = 8
FAST_V = 8


def _median_body(x_hbm, out_hbm, x_vmem, buf_vmem, out_vmem):
    c = lax.axis_index("c")
    s = lax.axis_index("s")
    wid = s * 2 + c
    row0 = wid * ROWS_PER_W

    kvec = jnp.full((L,), K, jnp.int32)
    zero = jnp.zeros((L,), jnp.int32)
    one = jnp.ones((L,), jnp.int32)
    top_bit = jnp.full((L,), -(2 ** 31), jnp.int32)
    mant_mask = jnp.full((L,), 0x7FFFFFFF, jnp.int32)
    pref_mask = jnp.full((L,), -(2 ** (32 - S1)), jnp.int32)
    nan_vec = jnp.full((L,), jnp.nan, jnp.float32)
    lane = lax.iota(jnp.int32, L)
    lane0 = lane == zero

    def key_to_float(key):
        fbits = jnp.where(key < zero, key & mant_mask, ~key)
        return lax.bitcast_convert_type(fbits, jnp.float32)

    def block_body(blk, carry):
        base_row = blk * RB
        pltpu.sync_copy(x_hbm.at[pl.ds((row0 + base_row) * N, RB * N)], x_vmem)

        def row_body(r, carry):
            off = r * N

            for j in range(FAST_V + 1):
                buf_vmem[pl.ds(j * L, L)] = nan_vec

            def bit1(_, st):
                prefix, bitv, t, below = st
                cand = prefix | bitv
                nbit = lax.shift_right_logical(bitv, 1)
                t_acc = key_to_float(cand | nbit)
                t_rej = key_to_float(prefix | nbit)

                def chunk(j, cnt):
                    e = off + j * (UNROLL * L)
                    for u in range(UNROLL):
                        v = x_vmem[pl.ds(e + u * L, L)]
                        cnt = cnt + plsc.all_reduce_population_count(v < t)
                    return cnt

                cnt = lax.fori_loop(0, N // (UNROLL * L), chunk, zero)
                acc = cnt <= kvec
                return (jnp.where(acc, cand, prefix), nbit,
                        jnp.where(acc, t_acc, t_rej),
                        jnp.where(acc, cnt, below))

            prefix, bitv, t, below = lax.fori_loop(
                0, S1, bit1, (zero, top_bit, key_to_float(top_bit), zero))
            krem = kvec - below

            def comp(j, o):
                for u in range(8):
                    e = off + (j * 8 + u) * L
                    v = x_vmem[pl.ds(e, L)]
                    b = lax.bitcast_convert_type(v + jnp.float32(0.0),
                                                 jnp.int32)
                    key = jnp.where(b < zero, ~b, b | top_bit)
                    m = (key & pref_mask) == prefix
                    csum = plsc.cumsum(jnp.where(m, one, zero))
                    idx = jnp.where(m, o + csum - one, zero)
                    plsc.store_scatter(buf_vmem, [idx], v, mask=m)
                    o = o + plsc.all_reduce_population_count(m)
                return o

            o = lax.fori_loop(0, N // (8 * L), comp, zero)
            plsc.store_scatter(buf_vmem, [o + lane], nan_vec)
            m_scalar = jnp.max(o)

            def bit2_fast(_, st):
                prefix, bitv, t = st
                cand = prefix | bitv
                nbit = lax.shift_right_logical(bitv, 1)
                t_acc = key_to_float(cand | nbit)
                t_rej = key_to_float(prefix | nbit)
                cnt = zero
                for j in range(FAST_V):
                    v = buf_vmem[pl.ds(j * L, L)]
                    cnt = cnt + plsc.all_reduce_population_count(v < t)
                acc = cnt <= krem
                return (jnp.where(acc, cand, prefix), nbit,
                        jnp.where(acc, t_acc, t_rej))

            def bit2_slow(_, st):
                prefix, bitv, t = st
                cand = prefix | bitv
                nbit = lax.shift_right_logical(bitv, 1)
                t_acc = key_to_float(cand | nbit)
                t_rej = key_to_float(prefix | nbit)
                nvreg = lax.shift_right_logical(m_scalar + (L - 1), 4)

                def chunk2(j, cnt):
                    v = buf_vmem[pl.ds(j * L, L)]
                    return cnt + plsc.all_reduce_population_count(v < t)

                cnt = lax.fori_loop(0, nvreg, chunk2, zero)
                acc = cnt <= krem
                return (jnp.where(acc, cand, prefix), nbit,
                        jnp.where(acc, t_acc, t_rej))

            def run_fast(st):
                return lax.fori_loop(0, 32 - S1, bit2_fast, st)[0]

            def run_slow(st):
                return lax.fori_loop(0, 32 - S1, bit2_slow, st)[0]

            prefix = lax.cond(m_scalar <= FAST_V * L, run_fast, run_slow,
                              (prefix, bitv, t))
            val = key_to_float(prefix)
            plsc.store_scatter(out_vmem,
                               [jnp.full((L,), base_row + r, jnp.int32)],
                               val, mask=lane0)
            return carry

        return lax.fori_loop(0, RB, row_body, carry)

    lax.fori_loop(0, ROWS_PER_W // RB, block_body, 0)
    pltpu.sync_copy(out_vmem, out_hbm.at[pl.ds(row0, ROWS_PER_W)])


def _median_call(xflat):
    return pl.kernel(
        _median_body,
        out_type=jax.ShapeDtypeStruct((ROWS,), jnp.float32),
        mesh=plsc.VectorSubcoreMesh(core_axis_name="c", subcore_axis_name="s"),
        scratch_types=[
            pltpu.VMEM((RB * N,), jnp.float32),
            pltpu.VMEM((N + L,), jnp.float32),
            pltpu.VMEM((ROWS_PER_W,), jnp.float32),
        ],
        compiler_params=pltpu.CompilerParams(needs_layout_passes=False),
    )(xflat)


@jax.jit
def kernel(x):
    out = _median_call(x.reshape(ROWS * N))
    return out.reshape(B0, B1)


# speedup vs baseline: 1.2500x; 1.1032x over previous
import jax
import jax.numpy as jnp
from jax import lax
from jax.experimental import pallas as pl
from jax.experimental.pallas import tpu as pltpu
from jax.experimental.pallas import tpu_sc as plsc

B0, B1, N = 4, 4096, 2048
ROWS = B0 * B1
K = (N - 1) // 2
NW = 32
ROWS_PER_W = ROWS // NW
RB = 32
L = 16
UNROLL = 8
S1 = 8
STRIDE = N // L + 1
FAST_P = 16


def _median_body(x_hbm, out_hbm, x_vmem, buf_vmem, out_vmem):
    c = lax.axis_index("c")
    s = lax.axis_index("s")
    wid = s * 2 + c
    row0 = wid * ROWS_PER_W

    kvec = jnp.full((L,), K, jnp.int32)
    zero = jnp.zeros((L,), jnp.int32)
    one = jnp.ones((L,), jnp.int32)
    top_bit = jnp.full((L,), -(2 ** 31), jnp.int32)
    mant_mask = jnp.full((L,), 0x7FFFFFFF, jnp.int32)
    pref_mask = jnp.full((L,), -(2 ** (32 - S1)), jnp.int32)
    lane = lax.iota(jnp.int32, L)
    lane0 = lane == zero
    lane_base = lane * jnp.full((L,), STRIDE, jnp.int32)

    def key_to_float(key):
        fbits = jnp.where(key < zero, key & mant_mask, ~key)
        return lax.bitcast_convert_type(fbits, jnp.float32)

    def block_body(blk, carry):
        base_row = blk * RB
        pltpu.sync_copy(x_hbm.at[pl.ds((row0 + base_row) * N, RB * N)], x_vmem)

        def row_body(r, carry):
            off = r * N


            def bit1(_, st):
                prefix, bitv, t, below = st
                cand = prefix | bitv
                nbit = lax.shift_right_logical(bitv, 1)
                t_acc = key_to_float(cand | nbit)
                t_rej = key_to_float(prefix | nbit)

                def chunk(j, cnt):
                    e = off + j * (UNROLL * L)
                    for u in range(UNROLL):
                        v = x_vmem[pl.ds(e + u * L, L)]
                        cnt = cnt + plsc.all_reduce_population_count(v < t)
                    return cnt

                cnt = lax.fori_loop(0, N // (UNROLL * L), chunk, zero)
                acc = cnt <= kvec
                return (jnp.where(acc, cand, prefix), nbit,
                        jnp.where(acc, t_acc, t_rej),
                        jnp.where(acc, cnt, below))

            prefix, bitv, t, below = lax.fori_loop(
                0, S1, bit1, (zero, top_bit, key_to_float(top_bit), zero))
            krem = kvec - below

            def comp(j, cnt_l):
                for u in range(8):
                    e = off + (j * 8 + u) * L
                    v = x_vmem[pl.ds(e, L)]
                    b = lax.bitcast_convert_type(v + jnp.float32(0.0),
                                                 jnp.int32)
                    key = jnp.where(b < zero, ~b, b | top_bit)
                    m = (key & pref_mask) == prefix
                    plsc.store_scatter(buf_vmem, [lane_base + cnt_l], v,
                                       mask=m)
                    cnt_l = cnt_l + jnp.where(m, one, zero)
                return cnt_l

            cnt_l = lax.fori_loop(0, N // (8 * L), comp, zero)
            maxc = jnp.max(cnt_l)

            def make_bit2(npos, unrolled):
                def bit2(_, st):
                    prefix, bitv, t = st
                    cand = prefix | bitv
                    nbit = lax.shift_right_logical(bitv, 1)
                    t_acc = key_to_float(cand | nbit)
                    t_rej = key_to_float(prefix | nbit)
                    if unrolled:
                        cnt = zero
                        for p in range(npos):
                            v = plsc.load_gather(
                                buf_vmem, [lane_base + jnp.full((L,), p,
                                                                jnp.int32)])
                            ok = (v < t) & (cnt_l > p)
                            cnt = cnt + plsc.all_reduce_population_count(ok)
                    else:
                        def chunk2(p, cnt):
                            pv = jnp.full((L,), p, jnp.int32)
                            v = plsc.load_gather(buf_vmem, [lane_base + pv])
                            ok = (v < t) & (cnt_l > pv)
                            return cnt + plsc.all_reduce_population_count(ok)

                        cnt = lax.fori_loop(0, npos, chunk2, zero)
                    acc = cnt <= krem
                    return (jnp.where(acc, cand, prefix), nbit,
                            jnp.where(acc, t_acc, t_rej))
                return bit2

            def run_fast(st):
                return lax.fori_loop(0, 32 - S1,
                                     make_bit2(FAST_P, True), st)[0]

            def run_slow(st):
                return lax.fori_loop(0, 32 - S1,
                                     make_bit2(maxc, False), st)[0]

            prefix = lax.cond(maxc <= FAST_P, run_fast, run_slow,
                              (prefix, bitv, t))
            val = key_to_float(prefix)
            plsc.store_scatter(out_vmem,
                               [jnp.full((L,), base_row + r, jnp.int32)],
                               val, mask=lane0)
            return carry

        return lax.fori_loop(0, RB, row_body, carry)

    lax.fori_loop(0, ROWS_PER_W // RB, block_body, 0)
    pltpu.sync_copy(out_vmem, out_hbm.at[pl.ds(row0, ROWS_PER_W)])


def _median_call(xflat):
    return pl.kernel(
        _median_body,
        out_type=jax.ShapeDtypeStruct((ROWS,), jnp.float32),
        mesh=plsc.VectorSubcoreMesh(core_axis_name="c", subcore_axis_name="s"),
        scratch_types=[
            pltpu.VMEM((RB * N,), jnp.float32),
            pltpu.VMEM((L * STRIDE,), jnp.float32),
            pltpu.VMEM((ROWS_PER_W,), jnp.float32),
        ],
        compiler_params=pltpu.CompilerParams(needs_layout_passes=False),
    )(xflat)


@jax.jit
def kernel(x):
    out = _median_call(x.reshape(ROWS * N))
    return out.reshape(B0, B1)


# speedup vs baseline: 2.1527x; 1.7222x over previous
import jax
import jax.numpy as jnp
from jax import lax
from jax.experimental import pallas as pl
from jax.experimental.pallas import tpu as pltpu
from jax.experimental.pallas import tpu_sc as plsc

B0, B1, N = 4, 4096, 2048
ROWS = B0 * B1
K = (N - 1) // 2
NW = 32
ROWS_PER_W = ROWS // NW
RB = 32
L = 16
UNROLL = 8
S1 = 8
STRIDE = N // L + 1
FAST_P = 16


def _median_body(x_hbm, out_hbm, x_vmem, buf_vmem, out_vmem):
    c = lax.axis_index("c")
    s = lax.axis_index("s")
    wid = s * 2 + c
    row0 = wid * ROWS_PER_W

    kvec = jnp.full((L,), K, jnp.int32)
    zero = jnp.zeros((L,), jnp.int32)
    one = jnp.ones((L,), jnp.int32)
    top_bit = jnp.full((L,), -(2 ** 31), jnp.int32)
    mant_mask = jnp.full((L,), 0x7FFFFFFF, jnp.int32)
    pref_mask = jnp.full((L,), -(2 ** (32 - S1)), jnp.int32)
    nan_vec = jnp.full((L,), jnp.nan, jnp.float32)
    lane = lax.iota(jnp.int32, L)
    lane0 = lane == zero
    lane_base = lane * jnp.full((L,), STRIDE, jnp.int32)

    def key_to_float(key):
        fbits = jnp.where(key < zero, key & mant_mask, ~key)
        return lax.bitcast_convert_type(fbits, jnp.float32)

    def block_body(blk, carry):
        base_row = blk * RB
        pltpu.sync_copy(x_hbm.at[pl.ds((row0 + base_row) * N, RB * N)], x_vmem)

        def row_body(r, carry):
            off = r * N


            def bit1(_, st):
                prefix, bitv, t, below = st
                cand = prefix | bitv
                nbit = lax.shift_right_logical(bitv, 1)
                t_acc = key_to_float(cand | nbit)
                t_rej = key_to_float(prefix | nbit)

                def chunk(j, cnt):
                    e = off + j * (UNROLL * L)
                    for u in range(UNROLL):
                        v = x_vmem[pl.ds(e + u * L, L)]
                        cnt = cnt + plsc.all_reduce_population_count(v < t)
                    return cnt

                cnt = lax.fori_loop(0, N // (UNROLL * L), chunk, zero)
                acc = cnt <= kvec
                return (jnp.where(acc, cand, prefix), nbit,
                        jnp.where(acc, t_acc, t_rej),
                        jnp.where(acc, cnt, below))

            prefix, bitv, t, below = lax.fori_loop(
                0, S1, bit1, (zero, top_bit, key_to_float(top_bit), zero))
            krem = kvec - below

            for p in range(FAST_P):
                plsc.store_scatter(buf_vmem,
                                   [lane_base + jnp.full((L,), p, jnp.int32)],
                                   nan_vec)

            def comp(j, cnt_l):
                vs, ms = [], []
                for u in range(8):
                    e = off + (j * 8 + u) * L
                    v = x_vmem[pl.ds(e, L)]
                    b = lax.bitcast_convert_type(v + jnp.float32(0.0),
                                                 jnp.int32)
                    key = jnp.where(b < zero, ~b, b | top_bit)
                    vs.append(v)
                    ms.append((key & pref_mask) == prefix)
                idxs = []
                for u in range(8):
                    idxs.append(lane_base + cnt_l)
                    cnt_l = cnt_l + jnp.where(ms[u], one, zero)
                for u in range(8):
                    plsc.store_scatter(buf_vmem, [idxs[u]], vs[u],
                                       mask=ms[u])
                return cnt_l

            cnt_l = lax.fori_loop(0, N // (8 * L), comp, zero)
            maxc = jnp.max(cnt_l)

            def make_bit2(npos, unrolled):
                def bit2(_, st):
                    prefix, bitv, t = st
                    cand = prefix | bitv
                    nbit = lax.shift_right_logical(bitv, 1)
                    t_acc = key_to_float(cand | nbit)
                    t_rej = key_to_float(prefix | nbit)
                    if unrolled:
                        cnt = zero
                        for p in range(npos):
                            v = plsc.load_gather(
                                buf_vmem, [lane_base + jnp.full((L,), p,
                                                                jnp.int32)])
                            cnt = cnt + plsc.all_reduce_population_count(
                                v < t)
                    else:
                        def chunk2(p, cnt):
                            pv = jnp.full((L,), p, jnp.int32)
                            v = plsc.load_gather(buf_vmem, [lane_base + pv])
                            ok = (v < t) & (cnt_l > pv)
                            return cnt + plsc.all_reduce_population_count(ok)

                        cnt = lax.fori_loop(0, npos, chunk2, zero)
                    acc = cnt <= krem
                    return (jnp.where(acc, cand, prefix), nbit,
                            jnp.where(acc, t_acc, t_rej))
                return bit2

            def run_fast(st):
                return lax.fori_loop(0, 32 - S1,
                                     make_bit2(FAST_P, True), st)[0]

            def run_slow(st):
                return lax.fori_loop(0, 32 - S1,
                                     make_bit2(maxc, False), st)[0]

            prefix = lax.cond(maxc <= FAST_P, run_fast, run_slow,
                              (prefix, bitv, t))
            val = key_to_float(prefix)
            plsc.store_scatter(out_vmem,
                               [jnp.full((L,), base_row + r, jnp.int32)],
                               val, mask=lane0)
            return carry

        return lax.fori_loop(0, RB, row_body, carry)

    lax.fori_loop(0, ROWS_PER_W // RB, block_body, 0)
    pltpu.sync_copy(out_vmem, out_hbm.at[pl.ds(row0, ROWS_PER_W)])


def _median_call(xflat):
    return pl.kernel(
        _median_body,
        out_type=jax.ShapeDtypeStruct((ROWS,), jnp.float32),
        mesh=plsc.VectorSubcoreMesh(core_axis_name="c", subcore_axis_name="s"),
        scratch_types=[
            pltpu.VMEM((RB * N,), jnp.float32),
            pltpu.VMEM((L * STRIDE,), jnp.float32),
            pltpu.VMEM((ROWS_PER_W,), jnp.float32),
        ],
        compiler_params=pltpu.CompilerParams(needs_layout_passes=False),
    )(xflat)


@jax.jit
def kernel(x):
    out = _median_call(x.reshape(ROWS * N))
    return out.reshape(B0, B1)


# speedup vs baseline: 2.2693x; 1.0542x over previous
import jax
import jax.numpy as jnp
from jax import lax
from jax.experimental import pallas as pl
from jax.experimental.pallas import tpu as pltpu
from jax.experimental.pallas import tpu_sc as plsc

B0, B1, N = 4, 4096, 2048
ROWS = B0 * B1
K = (N - 1) // 2
NW = 32
ROWS_PER_W = ROWS // NW
RB = 32
L = 16
UNROLL = 8
S1 = 8
STRIDE = N // L + 1
FAST_P = 16


def _median_body(x_hbm, out_hbm, x_vmem, buf_vmem, out_vmem):
    c = lax.axis_index("c")
    s = lax.axis_index("s")
    wid = s * 2 + c
    row0 = wid * ROWS_PER_W

    kvec = jnp.full((L,), K, jnp.int32)
    zero = jnp.zeros((L,), jnp.int32)
    one = jnp.ones((L,), jnp.int32)
    top_bit = jnp.full((L,), -(2 ** 31), jnp.int32)
    mant_mask = jnp.full((L,), 0x7FFFFFFF, jnp.int32)
    pref_mask = jnp.full((L,), -(2 ** (32 - S1)), jnp.int32)
    nan_vec = jnp.full((L,), jnp.nan, jnp.float32)
    lane = lax.iota(jnp.int32, L)
    lane0 = lane == zero
    lane_base = lane * jnp.full((L,), STRIDE, jnp.int32)

    def key_to_float(key):
        fbits = jnp.where(key < zero, key & mant_mask, ~key)
        return lax.bitcast_convert_type(fbits, jnp.float32)

    def block_body(blk, carry):
        base_row = blk * RB
        pltpu.sync_copy(x_hbm.at[pl.ds((row0 + base_row) * N, RB * N)], x_vmem)

        def row_body(r, carry):
            off = r * N


            def bit1(_, st):
                prefix, bitv, t, below = st
                cand = prefix | bitv
                nbit = lax.shift_right_logical(bitv, 1)
                t_acc = key_to_float(cand | nbit)
                t_rej = key_to_float(prefix | nbit)

                def chunk(j, cnt):
                    e = off + j * (UNROLL * L)
                    for u in range(UNROLL):
                        v = x_vmem[pl.ds(e + u * L, L)]
                        cnt = cnt + plsc.all_reduce_population_count(v < t)
                    return cnt

                cnt = lax.fori_loop(0, N // (UNROLL * L), chunk, zero)
                acc = cnt <= kvec
                return (jnp.where(acc, cand, prefix), nbit,
                        jnp.where(acc, t_acc, t_rej),
                        jnp.where(acc, cnt, below))

            prefix, bitv, t, below = lax.fori_loop(
                0, S1, bit1, (zero, top_bit, key_to_float(top_bit), zero))
            krem = kvec - below

            for p in range(FAST_P):
                plsc.store_scatter(buf_vmem,
                                   [lane_base + jnp.full((L,), p, jnp.int32)],
                                   nan_vec)

            lo = key_to_float(prefix)
            hi = key_to_float(prefix + jnp.full((L,), 2 ** (32 - S1),
                                                jnp.int32))
            is_top = prefix == pref_mask

            def comp(j, cnt_l):
                vs, ms = [], []
                for u in range(8):
                    e = off + (j * 8 + u) * L
                    v = x_vmem[pl.ds(e, L)]
                    vs.append(v)
                    ms.append(jnp.logical_not(v < lo)
                              & ((v < hi) | is_top))
                idxs = []
                for u in range(8):
                    idxs.append(lane_base + cnt_l)
                    cnt_l = cnt_l + jnp.where(ms[u], one, zero)
                for u in range(8):
                    plsc.store_scatter(buf_vmem, [idxs[u]], vs[u],
                                       mask=ms[u])
                return cnt_l

            cnt_l = lax.fori_loop(0, N // (8 * L), comp, zero)
            maxc = jnp.max(cnt_l)

            def make_bit2(npos, unrolled):
                def bit2(_, st):
                    prefix, bitv, t = st
                    cand = prefix | bitv
                    nbit = lax.shift_right_logical(bitv, 1)
                    t_acc = key_to_float(cand | nbit)
                    t_rej = key_to_float(prefix | nbit)
                    if unrolled:
                        cnt = zero
                        for p in range(npos):
                            v = plsc.load_gather(
                                buf_vmem, [lane_base + jnp.full((L,), p,
                                                                jnp.int32)])
                            cnt = cnt + plsc.all_reduce_population_count(
                                v < t)
                    else:
                        def chunk2(p, cnt):
                            pv = jnp.full((L,), p, jnp.int32)
                            v = plsc.load_gather(buf_vmem, [lane_base + pv])
                            ok = (v < t) & (cnt_l > pv)
                            return cnt + plsc.all_reduce_population_count(ok)

                        cnt = lax.fori_loop(0, npos, chunk2, zero)
                    acc = cnt <= krem
                    return (jnp.where(acc, cand, prefix), nbit,
                            jnp.where(acc, t_acc, t_rej))
                return bit2

            def run_fast(st):
                return lax.fori_loop(0, 32 - S1,
                                     make_bit2(FAST_P, True), st)[0]

            def run_slow(st):
                return lax.fori_loop(0, 32 - S1,
                                     make_bit2(maxc, False), st)[0]

            prefix = lax.cond(maxc <= FAST_P, run_fast, run_slow,
                              (prefix, bitv, t))
            val = key_to_float(prefix)
            plsc.store_scatter(out_vmem,
                               [jnp.full((L,), base_row + r, jnp.int32)],
                               val, mask=lane0)
            return carry

        return lax.fori_loop(0, RB, row_body, carry)

    lax.fori_loop(0, ROWS_PER_W // RB, block_body, 0)
    pltpu.sync_copy(out_vmem, out_hbm.at[pl.ds(row0, ROWS_PER_W)])


def _median_call(xflat):
    return pl.kernel(
        _median_body,
        out_type=jax.ShapeDtypeStruct((ROWS,), jnp.float32),
        mesh=plsc.VectorSubcoreMesh(core_axis_name="c", subcore_axis_name="s"),
        scratch_types=[
            pltpu.VMEM((RB * N,), jnp.float32),
            pltpu.VMEM((L * STRIDE,), jnp.float32),
            pltpu.VMEM((ROWS_PER_W,), jnp.float32),
        ],
        compiler_params=pltpu.CompilerParams(needs_layout_passes=False),
    )(xflat)


@jax.jit
def kernel(x):
    out = _median_call(x.reshape(ROWS * N))
    return out.reshape(B0, B1)


# speedup vs baseline: 2.2718x; 1.0011x over previous
import jax
import jax.numpy as jnp
from jax import lax
from jax.experimental import pallas as pl
from jax.experimental.pallas import tpu as pltpu
from jax.experimental.pallas import tpu_sc as plsc

B0, B1, N = 4, 4096, 2048
ROWS = B0 * B1
K = (N - 1) // 2
NW = 32
ROWS_PER_W = ROWS // NW
RB = 32
L = 16
UNROLL = 8
S1 = 8
STRIDE = N // L + 1
FAST_P = 16


def _median_body(x_hbm, out_hbm, x_vmem, buf_vmem, out_vmem):
    c = lax.axis_index("c")
    s = lax.axis_index("s")
    wid = s * 2 + c
    row0 = wid * ROWS_PER_W

    kvec = jnp.full((L,), K, jnp.int32)
    zero = jnp.zeros((L,), jnp.int32)
    one = jnp.ones((L,), jnp.int32)
    top_bit = jnp.full((L,), -(2 ** 31), jnp.int32)
    mant_mask = jnp.full((L,), 0x7FFFFFFF, jnp.int32)
    pref_mask = jnp.full((L,), -(2 ** (32 - S1)), jnp.int32)
    nan_vec = jnp.full((L,), jnp.nan, jnp.float32)
    lane = lax.iota(jnp.int32, L)
    lane0 = lane == zero
    lane_base = lane * jnp.full((L,), STRIDE, jnp.int32)

    def key_to_float(key):
        fbits = jnp.where(key < zero, key & mant_mask, ~key)
        return lax.bitcast_convert_type(fbits, jnp.float32)

    def block_body(blk, carry):
        base_row = blk * RB
        pltpu.sync_copy(x_hbm.at[pl.ds((row0 + base_row) * N, RB * N)], x_vmem)

        def row_body(r, carry):
            off = r * N


            def bit1(_, st):
                prefix, bitv, t, below = st
                cand = prefix | bitv
                nbit = lax.shift_right_logical(bitv, 1)
                t_acc = key_to_float(cand | nbit)
                t_rej = key_to_float(prefix | nbit)

                def chunk(j, cnt):
                    e = off + j * (UNROLL * L)
                    for u in range(UNROLL):
                        v = x_vmem[pl.ds(e + u * L, L)]
                        cnt = cnt + plsc.all_reduce_population_count(v < t)
                    return cnt

                cnt = lax.fori_loop(0, N // (UNROLL * L), chunk, zero)
                acc = cnt <= kvec
                return (jnp.where(acc, cand, prefix), nbit,
                        jnp.where(acc, t_acc, t_rej),
                        jnp.where(acc, cnt, below))

            prefix, bitv, t, below = lax.fori_loop(
                0, S1, bit1, (zero, top_bit, key_to_float(top_bit), zero))
            krem = kvec - below

            for p in range(FAST_P):
                plsc.store_scatter(buf_vmem,
                                   [lane_base + jnp.full((L,), p, jnp.int32)],
                                   nan_vec)

            lo = key_to_float(prefix)
            hi = key_to_float(prefix + jnp.full((L,), 2 ** (32 - S1),
                                                jnp.int32))
            is_top = prefix == pref_mask

            def comp(j, cnt_l):
                vs, ms = [], []
                for u in range(8):
                    e = off + (j * 8 + u) * L
                    v = x_vmem[pl.ds(e, L)]
                    vs.append(v)
                    ms.append(jnp.logical_not(v < lo)
                              & ((v < hi) | is_top))
                idxs = []
                for u in range(8):
                    idxs.append(lane_base + cnt_l)
                    cnt_l = cnt_l + jnp.where(ms[u], one, zero)
                for u in range(8):
                    plsc.store_scatter(buf_vmem, [idxs[u]], vs[u],
                                       mask=ms[u])
                return cnt_l

            cnt_l = lax.fori_loop(0, N // (8 * L), comp, zero)
            maxc = jnp.max(cnt_l)

            def finish_bit(st, cnt):
                prefix, bitv, t = st
                cand = prefix | bitv
                nbit = lax.shift_right_logical(bitv, 1)
                t_acc = key_to_float(cand | nbit)
                t_rej = key_to_float(prefix | nbit)
                acc = cnt <= krem
                return (jnp.where(acc, cand, prefix), nbit,
                        jnp.where(acc, t_acc, t_rej))

            def run_fast(st):
                ws = [plsc.load_gather(
                          buf_vmem,
                          [lane_base + jnp.full((L,), p, jnp.int32)])
                      for p in range(FAST_P)]

                def bit2(_, st):
                    t = st[2]
                    cnt = zero
                    for v in ws:
                        cnt = cnt + plsc.all_reduce_population_count(v < t)
                    return finish_bit(st, cnt)

                return lax.fori_loop(0, 32 - S1, bit2, st)[0]

            def run_slow(st):
                def bit2(_, st):
                    t = st[2]

                    def chunk2(p, cnt):
                        pv = jnp.full((L,), p, jnp.int32)
                        v = plsc.load_gather(buf_vmem, [lane_base + pv])
                        ok = (v < t) & (cnt_l > pv)
                        return cnt + plsc.all_reduce_population_count(ok)

                    cnt = lax.fori_loop(0, maxc, chunk2, zero)
                    return finish_bit(st, cnt)

                return lax.fori_loop(0, 32 - S1, bit2, st)[0]

            prefix = lax.cond(maxc <= FAST_P, run_fast, run_slow,
                              (prefix, bitv, t))
            val = key_to_float(prefix)
            plsc.store_scatter(out_vmem,
                               [jnp.full((L,), base_row + r, jnp.int32)],
                               val, mask=lane0)
            return carry

        return lax.fori_loop(0, RB, row_body, carry)

    lax.fori_loop(0, ROWS_PER_W // RB, block_body, 0)
    pltpu.sync_copy(out_vmem, out_hbm.at[pl.ds(row0, ROWS_PER_W)])


def _median_call(xflat):
    return pl.kernel(
        _median_body,
        out_type=jax.ShapeDtypeStruct((ROWS,), jnp.float32),
        mesh=plsc.VectorSubcoreMesh(core_axis_name="c", subcore_axis_name="s"),
        scratch_types=[
            pltpu.VMEM((RB * N,), jnp.float32),
            pltpu.VMEM((L * STRIDE,), jnp.float32),
            pltpu.VMEM((ROWS_PER_W,), jnp.float32),
        ],
        compiler_params=pltpu.CompilerParams(needs_layout_passes=False),
    )(xflat)


@jax.jit
def kernel(x):
    out = _median_call(x.reshape(ROWS * N))
    return out.reshape(B0, B1)


# speedup vs baseline: 2.4113x; 1.0614x over previous
import jax
import jax.numpy as jnp
from jax import lax
from jax.experimental import pallas as pl
from jax.experimental.pallas import tpu as pltpu
from jax.experimental.pallas import tpu_sc as plsc

B0, B1, N = 4, 4096, 2048
ROWS = B0 * B1
K = (N - 1) // 2
NW = 32
ROWS_PER_W = ROWS // NW
RB = 16
L = 16
UNROLL = 8
S1 = 8
STRIDE = N // L + 1
FAST_P = 16


def _median_body(x_hbm, out_hbm, x_vmem, x_vmem2, buf_vmem, out_vmem, sem_a,
                 sem_b):
    c = lax.axis_index("c")
    s = lax.axis_index("s")
    wid = s * 2 + c
    row0 = wid * ROWS_PER_W

    kvec = jnp.full((L,), K, jnp.int32)
    zero = jnp.zeros((L,), jnp.int32)
    one = jnp.ones((L,), jnp.int32)
    top_bit = jnp.full((L,), -(2 ** 31), jnp.int32)
    mant_mask = jnp.full((L,), 0x7FFFFFFF, jnp.int32)
    pref_mask = jnp.full((L,), -(2 ** (32 - S1)), jnp.int32)
    nan_vec = jnp.full((L,), jnp.nan, jnp.float32)
    lane = lax.iota(jnp.int32, L)
    lane0 = lane == zero
    lane_base = lane * jnp.full((L,), STRIDE, jnp.int32)

    def key_to_float(key):
        fbits = jnp.where(key < zero, key & mant_mask, ~key)
        return lax.bitcast_convert_type(fbits, jnp.float32)

    def hbm_slice(blk):
        return x_hbm.at[pl.ds((row0 + blk * RB) * N, RB * N)]

    def process_block(blk, xbuf):
        base_row = blk * RB

        def row_body(r, carry):
            off = r * N
            x_vmem = xbuf


            def bit1(_, st):
                prefix, bitv, t, below = st
                cand = prefix | bitv
                nbit = lax.shift_right_logical(bitv, 1)
                t_acc = key_to_float(cand | nbit)
                t_rej = key_to_float(prefix | nbit)

                def chunk(j, cnt):
                    e = off + j * (UNROLL * L)
                    for u in range(UNROLL):
                        v = x_vmem[pl.ds(e + u * L, L)]
                        cnt = cnt + plsc.all_reduce_population_count(v < t)
                    return cnt

                cnt = lax.fori_loop(0, N // (UNROLL * L), chunk, zero)
                acc = cnt <= kvec
                return (jnp.where(acc, cand, prefix), nbit,
                        jnp.where(acc, t_acc, t_rej),
                        jnp.where(acc, cnt, below))

            prefix, bitv, t, below = lax.fori_loop(
                0, S1, bit1, (zero, top_bit, key_to_float(top_bit), zero))
            krem = kvec - below

            for p in range(FAST_P):
                plsc.store_scatter(buf_vmem,
                                   [lane_base + jnp.full((L,), p, jnp.int32)],
                                   nan_vec)

            lo = key_to_float(prefix)
            hi = key_to_float(prefix + jnp.full((L,), 2 ** (32 - S1),
                                                jnp.int32))
            is_top = prefix == pref_mask

            def comp(j, cnt_l):
                vs, ms = [], []
                for u in range(8):
                    e = off + (j * 8 + u) * L
                    v = x_vmem[pl.ds(e, L)]
                    vs.append(v)
                    ms.append(jnp.logical_not(v < lo)
                              & ((v < hi) | is_top))
                idxs = []
                for u in range(8):
                    idxs.append(lane_base + cnt_l)
                    cnt_l = cnt_l + jnp.where(ms[u], one, zero)
                for u in range(8):
                    plsc.store_scatter(buf_vmem, [idxs[u]], vs[u],
                                       mask=ms[u])
                return cnt_l

            cnt_l = lax.fori_loop(0, N // (8 * L), comp, zero)
            maxc = jnp.max(cnt_l)

            def finish_bit(st, cnt):
                prefix, bitv, t = st
                cand = prefix | bitv
                nbit = lax.shift_right_logical(bitv, 1)
                t_acc = key_to_float(cand | nbit)
                t_rej = key_to_float(prefix | nbit)
                acc = cnt <= krem
                return (jnp.where(acc, cand, prefix), nbit,
                        jnp.where(acc, t_acc, t_rej))

            def run_fast(st):
                ws = [plsc.load_gather(
                          buf_vmem,
                          [lane_base + jnp.full((L,), p, jnp.int32)])
                      for p in range(FAST_P)]

                def bit2(_, st):
                    t = st[2]
                    cnt = zero
                    for v in ws:
                        cnt = cnt + plsc.all_reduce_population_count(v < t)
                    return finish_bit(st, cnt)

                return lax.fori_loop(0, 32 - S1, bit2, st)[0]

            def run_slow(st):
                def bit2(_, st):
                    t = st[2]

                    def chunk2(p, cnt):
                        pv = jnp.full((L,), p, jnp.int32)
                        v = plsc.load_gather(buf_vmem, [lane_base + pv])
                        ok = (v < t) & (cnt_l > pv)
                        return cnt + plsc.all_reduce_population_count(ok)

                    cnt = lax.fori_loop(0, maxc, chunk2, zero)
                    return finish_bit(st, cnt)

                return lax.fori_loop(0, 32 - S1, bit2, st)[0]

            prefix = lax.cond(maxc <= FAST_P, run_fast, run_slow,
                              (prefix, bitv, t))
            val = key_to_float(prefix)
            plsc.store_scatter(out_vmem,
                               [jnp.full((L,), base_row + r, jnp.int32)],
                               val, mask=lane0)
            return carry

        lax.fori_loop(0, RB, row_body, 0)

    NBLK = ROWS_PER_W // RB
    pltpu.async_copy(hbm_slice(0), x_vmem, sem_a)

    def pair_body(i, carry):
        blk_a = 2 * i
        pltpu.make_async_copy(hbm_slice(0), x_vmem, sem_a).wait()
        pltpu.async_copy(hbm_slice(blk_a + 1), x_vmem2, sem_b)
        process_block(blk_a, x_vmem)
        pltpu.make_async_copy(hbm_slice(0), x_vmem2, sem_b).wait()
        pltpu.async_copy(hbm_slice(jnp.minimum(blk_a + 2, NBLK - 1)),
                         x_vmem, sem_a)
        process_block(blk_a + 1, x_vmem2)
        return carry

    lax.fori_loop(0, NBLK // 2, pair_body, 0)
    pltpu.make_async_copy(hbm_slice(0), x_vmem, sem_a).wait()
    pltpu.sync_copy(out_vmem, out_hbm.at[pl.ds(row0, ROWS_PER_W)])


def _median_call(xflat):
    return pl.kernel(
        _median_body,
        out_type=jax.ShapeDtypeStruct((ROWS,), jnp.float32),
        mesh=plsc.VectorSubcoreMesh(core_axis_name="c", subcore_axis_name="s"),
        scratch_types=[
            pltpu.VMEM((RB * N,), jnp.float32),
            pltpu.VMEM((RB * N,), jnp.float32),
            pltpu.VMEM((L * STRIDE,), jnp.float32),
            pltpu.VMEM((ROWS_PER_W,), jnp.float32),
            pltpu.SemaphoreType.DMA,
            pltpu.SemaphoreType.DMA,
        ],
        compiler_params=pltpu.CompilerParams(needs_layout_passes=False),
    )(xflat)


@jax.jit
def kernel(x):
    out = _median_call(x.reshape(ROWS * N))
    return out.reshape(B0, B1)
